# speedup vs baseline: 13.0086x; 13.0086x over previous
import jax
import jax.numpy as jnp
from jax.experimental import pallas as pl

_N_PATCHES = 1024
_C_IN = 3
_KH = _KW = 5
_VALUES_DIM = 64
_K = 8
_STRIDE = 2
_PAD = 2
_KDIM = _C_IN * _KH * _KW
_KPAD = 128
_BR = 512


def _knn_block(a_ref, w_ref, b_ref, v_ref, o_ref):
    d = jnp.dot(a_ref[...], w_ref[...], preferred_element_type=jnp.float32)
    d = d + b_ref[...]
    iota = jax.lax.broadcasted_iota(jnp.int32, d.shape, 1)
    mask = jnp.zeros_like(d)
    for _ in range(_K):
        mn = jnp.min(d, axis=1, keepdims=True)
        am = jnp.min(jnp.where(d == mn, iota, _N_PATCHES), axis=1, keepdims=True)
        sel = iota == am
        mask = jnp.where(sel, 1.0, mask)
        d = jnp.where(sel, jnp.inf, d)
    o_ref[...] = jnp.dot(mask, v_ref[...],
                         preferred_element_type=jnp.float32) * (1.0 / _K)


@jax.jit
def kernel(x, kernel, bias, values):
    w = kernel
    b, c, h, _ = x.shape
    oh = ow = (h + 2 * _PAD - _KH) // _STRIDE + 1
    p = b * oh * ow

    xp = jnp.pad(x, ((0, 0), (0, 0), (_PAD, _PAD), (_PAD, _PAD)))
    span = _STRIDE * (oh - 1) + 1
    cols = [xp[:, :, ky:ky + span:_STRIDE, kx:kx + span:_STRIDE]
            for ky in range(_KH) for kx in range(_KW)]
    a = jnp.stack(cols, axis=2)
    a = a.transpose(0, 3, 4, 1, 2).reshape(p, _KDIM)
    a = jnp.pad(a, ((0, 0), (0, _KPAD - _KDIM)))

    wmat = jnp.pad(w.reshape(_N_PATCHES, _KDIM).T,
                   ((0, _KPAD - _KDIM), (0, 0)))
    brow = bias.reshape(1, _N_PATCHES)

    out = pl.pallas_call(
        _knn_block,
        grid=(p // _BR,),
        in_specs=[
            pl.BlockSpec((_BR, _KPAD), lambda i: (i, 0)),
            pl.BlockSpec((_KPAD, _N_PATCHES), lambda i: (0, 0)),
            pl.BlockSpec((1, _N_PATCHES), lambda i: (0, 0)),
            pl.BlockSpec((_N_PATCHES, _VALUES_DIM), lambda i: (0, 0)),
        ],
        out_specs=pl.BlockSpec((_BR, _VALUES_DIM), lambda i: (i, 0)),
        out_shape=jax.ShapeDtypeStruct((p, _VALUES_DIM), jnp.float32),
    )(a, wmat, brow, values)

    return out.reshape(b, oh, ow, _VALUES_DIM).transpose(0, 3, 1, 2)


# speedup vs baseline: 26.0080x; 1.9993x over previous
import jax
import jax.numpy as jnp
from jax.experimental import pallas as pl

_N_PATCHES = 1024
_C_IN = 3
_KH = _KW = 5
_VALUES_DIM = 64
_K = 8
_STRIDE = 2
_PAD = 2
_KDIM = _C_IN * _KH * _KW
_KPAD = 128
_BR = 512


def _knn_block(a_ref, w_ref, b_ref, v_ref, o_ref):
    d = jnp.dot(a_ref[...], w_ref[...], preferred_element_type=jnp.float32)
    d = d + b_ref[...]
    iota = jax.lax.broadcasted_iota(jnp.int32, d.shape, 1)
    mask = jnp.zeros_like(d)
    for _ in range(_K):
        mn = jnp.min(d, axis=1, keepdims=True)
        am = jnp.min(jnp.where(d == mn, iota, _N_PATCHES), axis=1, keepdims=True)
        sel = iota == am
        mask = jnp.where(sel, 1.0, mask)
        d = jnp.where(sel, jnp.inf, d)
    o_ref[...] = jnp.dot(mask, v_ref[...],
                         preferred_element_type=jnp.float32) * (1.0 / _K)


@jax.jit
def kernel(x, kernel, bias, values):
    w = kernel
    b, c, h, _ = x.shape
    oh = ow = (h + 2 * _PAD - _KH) // _STRIDE + 1
    p = b * oh * ow

    xp = jnp.pad(x, ((0, 0), (0, 0), (_PAD, _PAD), (_PAD, _PAD)))
    xv = xp.reshape(b, c, 228, 114, 2)
    xe = xv[..., 0].reshape(b, c, 114, 2, 114)
    xo = xv[..., 1].reshape(b, c, 114, 2, 114)
    cols = []
    for ky in range(_KH):
        for kx in range(_KW):
            src = xe if (kx % 2 == 0) else xo
            cols.append(src[:, :, (ky >> 1):(ky >> 1) + oh, ky & 1,
                            (kx >> 1):(kx >> 1) + ow])
    a = jnp.stack(cols, axis=2)
    a = a.transpose(0, 3, 4, 1, 2).reshape(p, _KDIM)
    a = jnp.pad(a, ((0, 0), (0, _KPAD - _KDIM)))

    wmat = jnp.pad(w.reshape(_N_PATCHES, _KDIM).T,
                   ((0, _KPAD - _KDIM), (0, 0)))
    brow = bias.reshape(1, _N_PATCHES)

    out = pl.pallas_call(
        _knn_block,
        grid=(p // _BR,),
        in_specs=[
            pl.BlockSpec((_BR, _KPAD), lambda i: (i, 0)),
            pl.BlockSpec((_KPAD, _N_PATCHES), lambda i: (0, 0)),
            pl.BlockSpec((1, _N_PATCHES), lambda i: (0, 0)),
            pl.BlockSpec((_N_PATCHES, _VALUES_DIM), lambda i: (0, 0)),
        ],
        out_specs=pl.BlockSpec((_BR, _VALUES_DIM), lambda i: (i, 0)),
        out_shape=jax.ShapeDtypeStruct((p, _VALUES_DIM), jnp.float32),
    )(a, wmat, brow, values)

    return out.reshape(b, oh, ow, _VALUES_DIM).transpose(0, 3, 1, 2)


# speedup vs baseline: 49.8689x; 1.9174x over previous
import jax
import jax.numpy as jnp
from jax.experimental import pallas as pl

_N_PATCHES = 1024
_C_IN = 3
_KH = _KW = 5
_VALUES_DIM = 64
_K = 8
_STRIDE = 2
_PAD = 2
_KDIM = _C_IN * _KH * _KW
_KPAD = 128
_LB = 512

_SORT8 = [(0, 1), (2, 3), (0, 2), (1, 3), (1, 2),
          (4, 5), (6, 7), (4, 6), (5, 7), (5, 6),
          (0, 4), (1, 5), (2, 6), (3, 7), (2, 4), (3, 5),
          (1, 2), (3, 4), (5, 6)]

_MERGE8 = [(0, 4), (1, 5), (2, 6), (3, 7),
           (0, 2), (1, 3), (4, 6), (5, 7),
           (0, 1), (2, 3), (4, 5), (6, 7)]


def _knn_block(w_ref, a_ref, b_ref, vt_ref, o_ref):
    d = jnp.dot(w_ref[...], a_ref[...], preferred_element_type=jnp.float32)
    d = d + b_ref[...]

    s = [d[k * 128:(k + 1) * 128, :] for k in range(8)]
    for i, j in _SORT8:
        lo = jnp.minimum(s[i], s[j])
        hi = jnp.maximum(s[i], s[j])
        s[i], s[j] = lo, hi

    half = 64
    while half >= 1:
        a = [t[:half, :] for t in s]
        bb = [t[half:2 * half, :] for t in s]
        m = [jnp.minimum(a[i], bb[7 - i]) for i in range(8)]
        for i, j in _MERGE8:
            lo = jnp.minimum(m[i], m[j])
            hi = jnp.maximum(m[i], m[j])
            m[i], m[j] = lo, hi
        s = m
        half //= 2

    t8 = s[7]

    maskf = jnp.where(d <= t8, 1.0, 0.0)
    cnt = jnp.sum(maskf, axis=0, keepdims=True)
    out = jnp.dot(vt_ref[...], maskf, preferred_element_type=jnp.float32)
    o_ref[...] = out / cnt


@jax.jit
def kernel(x, kernel, bias, values):
    w = kernel
    b, c, h, _ = x.shape
    oh = ow = (h + 2 * _PAD - _KH) // _STRIDE + 1
    p = b * oh * ow
    hp = (h + 2 * _PAD) // 2

    xp = jnp.pad(x, ((0, 0), (0, 0), (_PAD, _PAD), (_PAD, _PAD)))
    xv = xp.reshape(b, c, 2 * hp, hp, 2)
    xe = xv[..., 0].reshape(b, c, hp, 2, hp)
    xo = xv[..., 1].reshape(b, c, hp, 2, hp)
    rows = []
    for ci in range(c):
        for ky in range(_KH):
            for kx in range(_KW):
                src = xe if (kx % 2 == 0) else xo
                rows.append(src[:, ci, (ky >> 1):(ky >> 1) + oh, ky & 1,
                                (kx >> 1):(kx >> 1) + ow])
    at = jnp.stack(rows, axis=0).reshape(_KDIM, p)
    at = jnp.pad(at, ((0, _KPAD - _KDIM), (0, 0)))

    wmat = jnp.pad(w.reshape(_N_PATCHES, _KDIM),
                   ((0, 0), (0, _KPAD - _KDIM)))
    bcol = bias.reshape(_N_PATCHES, 1)
    vt = values.T

    out = pl.pallas_call(
        _knn_block,
        grid=(p // _LB,),
        in_specs=[
            pl.BlockSpec((_N_PATCHES, _KPAD), lambda i: (0, 0)),
            pl.BlockSpec((_KPAD, _LB), lambda i: (0, i)),
            pl.BlockSpec((_N_PATCHES, 1), lambda i: (0, 0)),
            pl.BlockSpec((_VALUES_DIM, _N_PATCHES), lambda i: (0, 0)),
        ],
        out_specs=pl.BlockSpec((_VALUES_DIM, _LB), lambda i: (0, i)),
        out_shape=jax.ShapeDtypeStruct((_VALUES_DIM, p), jnp.float32),
    )(wmat, at, bcol, vt)

    return out.reshape(_VALUES_DIM, b, oh, ow).transpose(1, 0, 2, 3)


# speedup vs baseline: 54.5512x; 1.0939x over previous
import jax
import jax.numpy as jnp
from jax.experimental import pallas as pl

_N_PATCHES = 1024
_C_IN = 3
_KH = _KW = 5
_VALUES_DIM = 64
_K = 8
_STRIDE = 2
_PAD = 2
_KDIM = _C_IN * _KH * _KW
_KPAD = 128
_LB = 512

_SORT8 = [(0, 1), (2, 3), (0, 2), (1, 3), (1, 2),
          (4, 5), (6, 7), (4, 6), (5, 7), (5, 6),
          (0, 4), (1, 5), (2, 6), (3, 7), (2, 4), (3, 5),
          (1, 2), (3, 4), (5, 6)]

_MERGE8 = [(0, 4), (1, 5), (2, 6), (3, 7),
           (0, 2), (1, 3), (4, 6), (5, 7),
           (0, 1), (2, 3), (4, 5), (6, 7)]


def _knn_block(w_ref, a_ref, b_ref, vt_ref, o_ref):
    d = jnp.dot(w_ref[...], a_ref[...], preferred_element_type=jnp.float32)
    d = d + b_ref[...]

    s = [d[k * 128:(k + 1) * 128, :] for k in range(8)]
    for i, j in _SORT8:
        lo = jnp.minimum(s[i], s[j])
        hi = jnp.maximum(s[i], s[j])
        s[i], s[j] = lo, hi

    half = 64
    while half >= 1:
        a = [t[:half, :] for t in s]
        bb = [t[half:2 * half, :] for t in s]
        m = [jnp.minimum(a[i], bb[7 - i]) for i in range(8)]
        for i, j in _MERGE8:
            lo = jnp.minimum(m[i], m[j])
            hi = jnp.maximum(m[i], m[j])
            m[i], m[j] = lo, hi
        s = m
        half //= 2

    t8 = s[7]

    maskf = jnp.where(d <= t8, 1.0, 0.0)
    cnt = jnp.sum(maskf, axis=0, keepdims=True)
    out = jnp.dot(vt_ref[...], maskf, preferred_element_type=jnp.float32)
    o_ref[...] = out / cnt


@jax.jit
def kernel(x, kernel, bias, values):
    w = kernel
    b, c, h, w_in = x.shape
    oh = ow = (h + 2 * _PAD - _KH) // _STRIDE + 1
    p = b * oh * ow
    hp = (h + 2 * _PAD) // 2

    owp = 128
    pw = b * oh * owp
    xp = jnp.pad(x, ((0, 0), (0, 0), (_PAD, _PAD),
                     (_PAD, 2 * (owp + 2) - w_in - _PAD)))
    xv = xp.reshape(b, c, 2 * hp, owp + 2, 2)
    xe = xv[..., 0].reshape(b, c, hp, 2, owp + 2)
    xo = xv[..., 1].reshape(b, c, hp, 2, owp + 2)
    rows = []
    for ci in range(c):
        for ky in range(_KH):
            for kx in range(_KW):
                src = xe if (kx % 2 == 0) else xo
                rows.append(src[:, ci, (ky >> 1):(ky >> 1) + oh, ky & 1,
                                (kx >> 1):(kx >> 1) + owp])
    at = jnp.stack(rows, axis=0).reshape(_KDIM, pw)

    wmat = w.reshape(_N_PATCHES, _KDIM)
    bcol = bias.reshape(_N_PATCHES, 1)
    vt = values.T

    out = pl.pallas_call(
        _knn_block,
        grid=(pw // _LB,),
        in_specs=[
            pl.BlockSpec((_N_PATCHES, _KDIM), lambda i: (0, 0)),
            pl.BlockSpec((_KDIM, _LB), lambda i: (0, i)),
            pl.BlockSpec((_N_PATCHES, 1), lambda i: (0, 0)),
            pl.BlockSpec((_VALUES_DIM, _N_PATCHES), lambda i: (0, 0)),
        ],
        out_specs=pl.BlockSpec((_VALUES_DIM, _LB), lambda i: (0, i)),
        out_shape=jax.ShapeDtypeStruct((_VALUES_DIM, pw), jnp.float32),
    )(wmat, at, bcol, vt)

    out = out.reshape(_VALUES_DIM, b, oh, owp)[..., :ow]
    return out.transpose(1, 0, 2, 3)


# speedup vs baseline: 73.4893x; 1.3472x over previous
import jax
import jax.numpy as jnp
from jax.experimental import pallas as pl

_N_PATCHES = 1024
_C_IN = 3
_KH = _KW = 5
_VALUES_DIM = 64
_K = 8
_STRIDE = 2
_PAD = 2
_KDIM = _C_IN * _KH * _KW
_OWP = 128
_RB = 4
_LB = _RB * _OWP

_SORT8 = [(0, 1), (2, 3), (0, 2), (1, 3), (1, 2),
          (4, 5), (6, 7), (4, 6), (5, 7), (5, 6),
          (0, 4), (1, 5), (2, 6), (3, 7), (2, 4), (3, 5),
          (1, 2), (3, 4), (5, 6)]

_MERGE8 = [(0, 4), (1, 5), (2, 6), (3, 7),
           (0, 2), (1, 3), (4, 6), (5, 7),
           (0, 1), (2, 3), (4, 5), (6, 7)]


def _knn_block(w_ref, xe_ref, xo_ref, b_ref, vt_ref, o_ref):
    iy = pl.program_id(1)
    groups = []
    for r in range(_RB):
        rows = []
        for ky in range(_KH):
            for kx in range(_KW):
                src = xe_ref if kx % 2 == 0 else xo_ref
                row_idx = iy * _RB + r + (ky >> 1)
                t = src[0, :, pl.ds(row_idx, 1), ky & 1,
                        pl.ds(kx >> 1, _OWP)]
                rows.append(t.reshape(_C_IN, _OWP))
        xg = jnp.concatenate(rows, axis=0)
        groups.append(jnp.dot(w_ref[...], xg,
                              preferred_element_type=jnp.float32))
    d = jnp.concatenate(groups, axis=1) + b_ref[...]

    s = [d[k * 128:(k + 1) * 128, :] for k in range(8)]
    for i, j in _SORT8:
        lo = jnp.minimum(s[i], s[j])
        hi = jnp.maximum(s[i], s[j])
        s[i], s[j] = lo, hi

    half = 64
    while half >= 1:
        a = [t[:half, :] for t in s]
        bb = [t[half:2 * half, :] for t in s]
        m = [jnp.minimum(a[i], bb[7 - i]) for i in range(8)]
        for i, j in _MERGE8:
            lo = jnp.minimum(m[i], m[j])
            hi = jnp.maximum(m[i], m[j])
            m[i], m[j] = lo, hi
        s = m
        half //= 2

    t8 = s[7]

    maskf = jnp.where(d <= t8, 1.0, 0.0)
    cnt = jnp.sum(maskf, axis=0, keepdims=True)
    out = jnp.dot(vt_ref[...], maskf, preferred_element_type=jnp.float32)
    o_ref[...] = out / cnt


@jax.jit
def kernel(x, kernel, bias, values):
    w = kernel
    b, c, h, w_in = x.shape
    oh = ow = (h + 2 * _PAD - _KH) // _STRIDE + 1
    hp = (h + 2 * _PAD) // 2
    pw = b * oh * _OWP

    xp = jnp.pad(x, ((0, 0), (0, 0), (_PAD, _PAD),
                     (_PAD, 2 * (_OWP + 2) - w_in - _PAD)))
    xv = xp.reshape(b, c, 2 * hp, _OWP + 2, 2)
    xe = xv[..., 0].reshape(b, c, hp, 2, _OWP + 2)
    xo = xv[..., 1].reshape(b, c, hp, 2, _OWP + 2)

    wmat = w.reshape(_N_PATCHES, _C_IN, _KH * _KW).transpose(0, 2, 1)
    wmat = wmat.reshape(_N_PATCHES, _KDIM)
    bcol = bias.reshape(_N_PATCHES, 1)
    vt = values.T

    out = pl.pallas_call(
        _knn_block,
        grid=(b, oh // _RB),
        in_specs=[
            pl.BlockSpec((_N_PATCHES, _KDIM), lambda ib, iy: (0, 0)),
            pl.BlockSpec((1, c, hp, 2, _OWP + 2), lambda ib, iy: (ib, 0, 0, 0, 0)),
            pl.BlockSpec((1, c, hp, 2, _OWP + 2), lambda ib, iy: (ib, 0, 0, 0, 0)),
            pl.BlockSpec((_N_PATCHES, 1), lambda ib, iy: (0, 0)),
            pl.BlockSpec((_VALUES_DIM, _N_PATCHES), lambda ib, iy: (0, 0)),
        ],
        out_specs=pl.BlockSpec((_VALUES_DIM, _LB),
                               lambda ib, iy: (0, ib * (oh // _RB) + iy)),
        out_shape=jax.ShapeDtypeStruct((_VALUES_DIM, pw), jnp.float32),
    )(wmat, xe, xo, bcol, vt)

    out = out.reshape(_VALUES_DIM, b, oh, _OWP)[..., :ow]
    return out.transpose(1, 0, 2, 3)


# speedup vs baseline: 73.5839x; 1.0013x over previous
import jax
import jax.numpy as jnp
from jax.experimental import pallas as pl
from jax.experimental.pallas import tpu as pltpu

_N_PATCHES = 1024
_C_IN = 3
_KH = _KW = 5
_VALUES_DIM = 64
_K = 8
_STRIDE = 2
_PAD = 2
_KDIM = _C_IN * _KH * _KW
_OWP = 128
_RB = 4
_LB = _RB * _OWP

_SORT8 = [(0, 1), (2, 3), (0, 2), (1, 3), (1, 2),
          (4, 5), (6, 7), (4, 6), (5, 7), (5, 6),
          (0, 4), (1, 5), (2, 6), (3, 7), (2, 4), (3, 5),
          (1, 2), (3, 4), (5, 6)]

_MERGE8 = [(0, 4), (1, 5), (2, 6), (3, 7),
           (0, 2), (1, 3), (4, 6), (5, 7),
           (0, 1), (2, 3), (4, 5), (6, 7)]


def _knn_block(w_ref, xe_ref, xo_ref, b_ref, vt_ref, o_ref):
    iy = pl.program_id(1)
    groups = []
    for r in range(_RB):
        rows = []
        for ky in range(_KH):
            for kx in range(_KW):
                src = xe_ref if kx % 2 == 0 else xo_ref
                row_idx = iy * _RB + r + (ky >> 1)
                t = src[0, :, pl.ds(row_idx, 1), ky & 1,
                        pl.ds(kx >> 1, _OWP)]
                rows.append(t.reshape(_C_IN, _OWP))
        xg = jnp.concatenate(rows, axis=0)
        groups.append(jnp.dot(w_ref[...], xg,
                              preferred_element_type=jnp.float32))
    d = jnp.concatenate(groups, axis=1) + b_ref[...]

    s = [d[k * 128:(k + 1) * 128, :] for k in range(8)]
    for i, j in _SORT8:
        lo = jnp.minimum(s[i], s[j])
        hi = jnp.maximum(s[i], s[j])
        s[i], s[j] = lo, hi

    half = 64
    while half >= 1:
        a = [t[:half, :] for t in s]
        bb = [t[half:2 * half, :] for t in s]
        m = [jnp.minimum(a[i], bb[7 - i]) for i in range(8)]
        for i, j in _MERGE8:
            lo = jnp.minimum(m[i], m[j])
            hi = jnp.maximum(m[i], m[j])
            m[i], m[j] = lo, hi
        s = m
        half //= 2

    t8 = s[7]

    maskf = jnp.where(d <= t8, 1.0, 0.0)
    cnt = jnp.sum(maskf, axis=0, keepdims=True)
    out = jnp.dot(vt_ref[...], maskf, preferred_element_type=jnp.float32)
    o_ref[...] = out / cnt


@jax.jit
def kernel(x, kernel, bias, values):
    w = kernel
    b, c, h, w_in = x.shape
    oh = ow = (h + 2 * _PAD - _KH) // _STRIDE + 1
    hp = (h + 2 * _PAD) // 2
    pw = b * oh * _OWP

    xp = jnp.pad(x, ((0, 0), (0, 0), (_PAD, _PAD),
                     (_PAD, 2 * (_OWP + 2) - w_in - _PAD)))
    xv = xp.reshape(b, c, 2 * hp, _OWP + 2, 2)
    xe = xv[..., 0].reshape(b, c, hp, 2, _OWP + 2)
    xo = xv[..., 1].reshape(b, c, hp, 2, _OWP + 2)

    wmat = w.reshape(_N_PATCHES, _C_IN, _KH * _KW).transpose(0, 2, 1)
    wmat = wmat.reshape(_N_PATCHES, _KDIM)
    bcol = bias.reshape(_N_PATCHES, 1)
    vt = values.T

    out = pl.pallas_call(
        _knn_block,
        grid=(b, oh // _RB),
        in_specs=[
            pl.BlockSpec((_N_PATCHES, _KDIM), lambda ib, iy: (0, 0)),
            pl.BlockSpec((1, c, hp, 2, _OWP + 2), lambda ib, iy: (ib, 0, 0, 0, 0)),
            pl.BlockSpec((1, c, hp, 2, _OWP + 2), lambda ib, iy: (ib, 0, 0, 0, 0)),
            pl.BlockSpec((_N_PATCHES, 1), lambda ib, iy: (0, 0)),
            pl.BlockSpec((_VALUES_DIM, _N_PATCHES), lambda ib, iy: (0, 0)),
        ],
        out_specs=pl.BlockSpec((_VALUES_DIM, _LB),
                               lambda ib, iy: (0, ib * (oh // _RB) + iy)),
        out_shape=jax.ShapeDtypeStruct((_VALUES_DIM, pw), jnp.float32),
        compiler_params=pltpu.CompilerParams(
            dimension_semantics=("parallel", "parallel")),
    )(wmat, xe, xo, bcol, vt)

    out = out.reshape(_VALUES_DIM, b, oh, _OWP)[..., :ow]
    return out.transpose(1, 0, 2, 3)


# speedup vs baseline: 73.7797x; 1.0027x over previous
import jax
import jax.numpy as jnp
from jax.experimental import pallas as pl
from jax.experimental.pallas import tpu as pltpu

_N_PATCHES = 1024
_C_IN = 3
_KH = _KW = 5
_VALUES_DIM = 64
_K = 8
_STRIDE = 2
_PAD = 2
_KDIM = _C_IN * _KH * _KW
_OWP = 128
_RB = 4
_LB = _RB * _OWP

_SORT8 = [(0, 1), (2, 3), (0, 2), (1, 3), (1, 2),
          (4, 5), (6, 7), (4, 6), (5, 7), (5, 6),
          (0, 4), (1, 5), (2, 6), (3, 7), (2, 4), (3, 5),
          (1, 2), (3, 4), (5, 6)]

_MERGE8 = [(0, 4), (1, 5), (2, 6), (3, 7),
           (0, 2), (1, 3), (4, 6), (5, 7),
           (0, 1), (2, 3), (4, 5), (6, 7)]


def _knn_block(w_ref, xe_ref, xo_ref, b_ref, vt_ref, o_ref):
    iy = pl.program_id(1)
    rows = []
    for ky in range(_KH):
        for kx in range(_KW):
            src = xe_ref if kx % 2 == 0 else xo_ref
            segs = [src[0, :, pl.ds(iy * _RB + r + (ky >> 1), 1), ky & 1,
                        pl.ds(kx >> 1, _OWP)].reshape(_C_IN, _OWP)
                    for r in range(_RB)]
            rows.append(jnp.concatenate(segs, axis=1))
    xg = jnp.concatenate(rows, axis=0)
    d = jnp.dot(w_ref[...], xg,
                preferred_element_type=jnp.float32) + b_ref[...]

    s = [d[k * 128:(k + 1) * 128, :] for k in range(8)]
    for i, j in _SORT8:
        lo = jnp.minimum(s[i], s[j])
        hi = jnp.maximum(s[i], s[j])
        s[i], s[j] = lo, hi

    half = 64
    while half >= 1:
        a = [t[:half, :] for t in s]
        bb = [t[half:2 * half, :] for t in s]
        m = [jnp.minimum(a[i], bb[7 - i]) for i in range(8)]
        for i, j in _MERGE8:
            lo = jnp.minimum(m[i], m[j])
            hi = jnp.maximum(m[i], m[j])
            m[i], m[j] = lo, hi
        s = m
        half //= 2

    t8 = s[7]

    maskf = jnp.where(d <= t8, 1.0, 0.0)
    cnt = jnp.sum(maskf, axis=0, keepdims=True)
    out = jnp.dot(vt_ref[...], maskf, preferred_element_type=jnp.float32)
    o_ref[...] = out / cnt


@jax.jit
def kernel(x, kernel, bias, values):
    w = kernel
    b, c, h, w_in = x.shape
    oh = ow = (h + 2 * _PAD - _KH) // _STRIDE + 1
    hp = (h + 2 * _PAD) // 2
    pw = b * oh * _OWP

    xp = jnp.pad(x, ((0, 0), (0, 0), (_PAD, _PAD),
                     (_PAD, 2 * (_OWP + 2) - w_in - _PAD)))
    xv = xp.reshape(b, c, 2 * hp, _OWP + 2, 2)
    xe = xv[..., 0].reshape(b, c, hp, 2, _OWP + 2)
    xo = xv[..., 1].reshape(b, c, hp, 2, _OWP + 2)

    wmat = w.reshape(_N_PATCHES, _C_IN, _KH * _KW).transpose(0, 2, 1)
    wmat = wmat.reshape(_N_PATCHES, _KDIM)
    bcol = bias.reshape(_N_PATCHES, 1)
    vt = values.T

    out = pl.pallas_call(
        _knn_block,
        grid=(b, oh // _RB),
        in_specs=[
            pl.BlockSpec((_N_PATCHES, _KDIM), lambda ib, iy: (0, 0)),
            pl.BlockSpec((1, c, hp, 2, _OWP + 2), lambda ib, iy: (ib, 0, 0, 0, 0)),
            pl.BlockSpec((1, c, hp, 2, _OWP + 2), lambda ib, iy: (ib, 0, 0, 0, 0)),
            pl.BlockSpec((_N_PATCHES, 1), lambda ib, iy: (0, 0)),
            pl.BlockSpec((_VALUES_DIM, _N_PATCHES), lambda ib, iy: (0, 0)),
        ],
        out_specs=pl.BlockSpec((_VALUES_DIM, _LB),
                               lambda ib, iy: (0, ib * (oh // _RB) + iy)),
        out_shape=jax.ShapeDtypeStruct((_VALUES_DIM, pw), jnp.float32),
        compiler_params=pltpu.CompilerParams(
            dimension_semantics=("parallel", "parallel")),
    )(wmat, xe, xo, bcol, vt)

    out = out.reshape(_VALUES_DIM, b, oh, _OWP)[..., :ow]
    return out.transpose(1, 0, 2, 3)
